# baseline (speedup 1.0000x reference)
import functools

import jax
import jax.numpy as jnp
from jax import lax
from jax.experimental import pallas as pl
from jax.experimental.pallas import tpu as pltpu
from jax.experimental.pallas import tpu_sc as plsc

_MAX_LEN = 200
_OBS = 128
_NC = 2
_NS = 16
_LANES = 16


def _zero_body(buf_ref, mask_ref):
    buf_ref[...] = jnp.zeros_like(buf_ref)
    pos = lax.broadcasted_iota(jnp.int32, mask_ref.shape, 1)
    mask_ref[...] = (pos >= _MAX_LEN - 1).astype(jnp.int32)


def _zero_fill(batch, dtype):
    bblk = 128
    return pl.pallas_call(
        _zero_body,
        grid=(batch // bblk,),
        in_specs=[],
        out_specs=[
            pl.BlockSpec((bblk, _MAX_LEN, _OBS), lambda i: (i, 0, 0)),
            pl.BlockSpec((1, _MAX_LEN), lambda i: (0, 0)),
        ],
        out_shape=[
            jax.ShapeDtypeStruct((batch, _MAX_LEN, _OBS), dtype),
            jax.ShapeDtypeStruct((1, _MAX_LEN), jnp.int32),
        ],
        compiler_params=pltpu.CompilerParams(
            dimension_semantics=("parallel",),
        ),
    )()


def _make_sc_scatter(batch):
    nw = _NC * _NS
    b_per_w = batch // nw
    mesh = plsc.VectorSubcoreMesh(
        core_axis_name="c", subcore_axis_name="s",
        num_cores=_NC, num_subcores=_NS,
    )

    @functools.partial(
        pl.kernel,
        mesh=mesh,
        out_type=(),
        scratch_types=[
            pltpu.VMEM((b_per_w, _OBS), jnp.float32),
            pltpu.VMEM((b_per_w,), jnp.int32),
            pltpu.SemaphoreType.DMA,
            pltpu.SemaphoreType.DMA,
        ],
    )
    def sc_scatter(x_hbm, buf_ref, rows_v, idx_v, sem, sem2):
        wid = lax.axis_index("s") * _NC + lax.axis_index("c")
        base = wid * b_per_w
        stage = pltpu.async_copy(x_hbm.at[pl.ds(base, b_per_w)], rows_v, sem2)
        lane = lax.iota(jnp.int32, _LANES)
        for j in range(b_per_w // _LANES):
            idx_v[pl.ds(j * _LANES, _LANES)] = (
                base + j * _LANES + lane) * _MAX_LEN
        stage.wait()
        pltpu.async_copy(rows_v, buf_ref.at[idx_v], sem).wait()

    return sc_scatter


def kernel(x):
    batch = x.shape[0]
    buf, mask = _zero_fill(batch, x.dtype)
    buf_ref = jax.new_ref(buf.reshape(batch * _MAX_LEN, _OBS))
    _make_sc_scatter(batch)(x, buf_ref)
    out = buf_ref[...].reshape(batch, _MAX_LEN, _OBS)
    return out, (mask[0] != 0)


# speedup vs baseline: 1.0126x; 1.0126x over previous
import functools

import jax
import jax.numpy as jnp
from jax import lax
from jax.experimental import pallas as pl
from jax.experimental.pallas import tpu as pltpu
from jax.experimental.pallas import tpu_sc as plsc

_MAX_LEN = 200
_OBS = 128
_NC = 2
_NS = 16
_LANES = 16


def _zero_body(buf_ref, mask_ref):
    buf_ref[...] = jnp.zeros_like(buf_ref)
    pos = lax.broadcasted_iota(jnp.int32, mask_ref.shape, 1)
    mask_ref[...] = (pos >= _MAX_LEN - 1).astype(jnp.int32)


def _zero_fill(batch, dtype):
    bblk = 64
    return pl.pallas_call(
        _zero_body,
        grid=(batch // bblk,),
        in_specs=[],
        out_specs=[
            pl.BlockSpec((bblk, _MAX_LEN, _OBS), lambda i: (i, 0, 0)),
            pl.BlockSpec((1, _MAX_LEN), lambda i: (0, 0)),
        ],
        out_shape=[
            jax.ShapeDtypeStruct((batch, _MAX_LEN, _OBS), dtype),
            jax.ShapeDtypeStruct((1, _MAX_LEN), jnp.int32),
        ],
        compiler_params=pltpu.CompilerParams(
            dimension_semantics=("parallel",),
        ),
    )()


def _make_sc_scatter(batch):
    nw = _NC * _NS
    b_per_w = batch // nw
    mesh = plsc.VectorSubcoreMesh(
        core_axis_name="c", subcore_axis_name="s",
        num_cores=_NC, num_subcores=_NS,
    )

    @functools.partial(
        pl.kernel,
        mesh=mesh,
        out_type=(),
        scratch_types=[
            pltpu.VMEM((b_per_w, _OBS), jnp.float32),
            pltpu.VMEM((b_per_w,), jnp.int32),
            pltpu.SemaphoreType.DMA,
            pltpu.SemaphoreType.DMA,
        ],
    )
    def sc_scatter(x_hbm, buf_ref, rows_v, idx_v, sem, sem2):
        wid = lax.axis_index("s") * _NC + lax.axis_index("c")
        base = wid * b_per_w
        stage = pltpu.async_copy(x_hbm.at[pl.ds(base, b_per_w)], rows_v, sem2)
        lane = lax.iota(jnp.int32, _LANES)
        for j in range(b_per_w // _LANES):
            idx_v[pl.ds(j * _LANES, _LANES)] = (
                base + j * _LANES + lane) * _MAX_LEN
        stage.wait()
        pltpu.async_copy(rows_v, buf_ref.at[idx_v], sem).wait()

    return sc_scatter


def kernel(x):
    batch = x.shape[0]
    buf, mask = _zero_fill(batch, x.dtype)
    buf_ref = jax.new_ref(buf.reshape(batch * _MAX_LEN, _OBS))
    _make_sc_scatter(batch)(x, buf_ref)
    out = buf_ref[...].reshape(batch, _MAX_LEN, _OBS)
    return out, (mask[0] != 0)


# speedup vs baseline: 1.0128x; 1.0002x over previous
import functools

import jax
import jax.numpy as jnp
from jax import lax
from jax.experimental import pallas as pl
from jax.experimental.pallas import tpu as pltpu
from jax.experimental.pallas import tpu_sc as plsc

_MAX_LEN = 200
_OBS = 128
_NC = 2
_NS = 16
_LANES = 16


def _zero_body(buf_ref, mask_ref):
    buf_ref[...] = jnp.zeros_like(buf_ref)
    pos = lax.broadcasted_iota(jnp.int32, mask_ref.shape, 1)
    mask_ref[...] = pos >= _MAX_LEN - 1


def _zero_fill(batch, dtype):
    bblk = 64
    return pl.pallas_call(
        _zero_body,
        grid=(batch // bblk,),
        in_specs=[],
        out_specs=[
            pl.BlockSpec((bblk, _MAX_LEN, _OBS), lambda i: (i, 0, 0)),
            pl.BlockSpec((1, _MAX_LEN), lambda i: (0, 0)),
        ],
        out_shape=[
            jax.ShapeDtypeStruct((batch, _MAX_LEN, _OBS), dtype),
            jax.ShapeDtypeStruct((1, _MAX_LEN), jnp.bool_),
        ],
        compiler_params=pltpu.CompilerParams(
            dimension_semantics=("parallel",),
        ),
    )()


def _make_sc_scatter(batch):
    nw = _NC * _NS
    b_per_w = batch // nw
    mesh = plsc.VectorSubcoreMesh(
        core_axis_name="c", subcore_axis_name="s",
        num_cores=_NC, num_subcores=_NS,
    )

    @functools.partial(
        pl.kernel,
        mesh=mesh,
        out_type=(),
        scratch_types=[
            pltpu.VMEM((b_per_w, _OBS), jnp.float32),
            pltpu.VMEM((b_per_w,), jnp.int32),
            pltpu.SemaphoreType.DMA,
            pltpu.SemaphoreType.DMA,
        ],
    )
    def sc_scatter(x_hbm, buf_ref, rows_v, idx_v, sem, sem2):
        wid = lax.axis_index("s") * _NC + lax.axis_index("c")
        base = wid * b_per_w
        stage = pltpu.async_copy(x_hbm.at[pl.ds(base, b_per_w)], rows_v, sem2)
        lane = lax.iota(jnp.int32, _LANES)
        for j in range(b_per_w // _LANES):
            idx_v[pl.ds(j * _LANES, _LANES)] = (
                base + j * _LANES + lane) * _MAX_LEN
        stage.wait()
        pltpu.async_copy(rows_v, buf_ref.at[idx_v], sem).wait()

    return sc_scatter


def kernel(x):
    batch = x.shape[0]
    buf, mask = _zero_fill(batch, x.dtype)
    buf_ref = jax.new_ref(buf.reshape(batch * _MAX_LEN, _OBS))
    _make_sc_scatter(batch)(x, buf_ref)
    out = buf_ref[...].reshape(batch, _MAX_LEN, _OBS)
    return out, mask[0]


# speedup vs baseline: 1.0420x; 1.0288x over previous
import functools

import jax
import jax.numpy as jnp
from jax import lax
from jax.experimental import pallas as pl
from jax.experimental.pallas import tpu as pltpu
from jax.experimental.pallas import tpu_sc as plsc

_MAX_LEN = 200
_OBS = 128
_NC = 2
_NS = 16
_LANES = 16


def _zero_body(buf_ref, mask_ref):
    buf_ref[...] = jnp.zeros_like(buf_ref)
    pos = lax.broadcasted_iota(jnp.int32, mask_ref.shape, 1)
    mask_ref[...] = pos >= _MAX_LEN - 1


def _zero_fill(batch, dtype):
    bblk = 32
    return pl.pallas_call(
        _zero_body,
        grid=(batch // bblk,),
        in_specs=[],
        out_specs=[
            pl.BlockSpec((bblk, _MAX_LEN, _OBS), lambda i: (i, 0, 0)),
            pl.BlockSpec((1, _MAX_LEN), lambda i: (0, 0)),
        ],
        out_shape=[
            jax.ShapeDtypeStruct((batch, _MAX_LEN, _OBS), dtype),
            jax.ShapeDtypeStruct((1, _MAX_LEN), jnp.bool_),
        ],
        compiler_params=pltpu.CompilerParams(
            dimension_semantics=("parallel",),
        ),
    )()


def _make_sc_scatter(batch):
    nw = _NC * _NS
    b_per_w = batch // nw
    mesh = plsc.VectorSubcoreMesh(
        core_axis_name="c", subcore_axis_name="s",
        num_cores=_NC, num_subcores=_NS,
    )

    @functools.partial(
        pl.kernel,
        mesh=mesh,
        out_type=(),
        scratch_types=[
            pltpu.VMEM((b_per_w, _OBS), jnp.float32),
            pltpu.VMEM((b_per_w,), jnp.int32),
            pltpu.SemaphoreType.DMA,
            pltpu.SemaphoreType.DMA,
        ],
    )
    def sc_scatter(x_hbm, buf_ref, rows_v, idx_v, sem, sem2):
        wid = lax.axis_index("s") * _NC + lax.axis_index("c")
        base = wid * b_per_w
        stage = pltpu.async_copy(x_hbm.at[pl.ds(base, b_per_w)], rows_v, sem2)
        lane = lax.iota(jnp.int32, _LANES)
        for j in range(b_per_w // _LANES):
            idx_v[pl.ds(j * _LANES, _LANES)] = (
                base + j * _LANES + lane) * _MAX_LEN
        stage.wait()
        pltpu.async_copy(rows_v, buf_ref.at[idx_v], sem).wait()

    return sc_scatter


def kernel(x):
    batch = x.shape[0]
    buf, mask = _zero_fill(batch, x.dtype)
    buf_ref = jax.new_ref(buf.reshape(batch * _MAX_LEN, _OBS))
    _make_sc_scatter(batch)(x, buf_ref)
    out = buf_ref[...].reshape(batch, _MAX_LEN, _OBS)
    return out, mask[0]
